# Initial kernel scaffold; baseline (speedup 1.0000x reference)
import jax
import jax.numpy as jnp
from jax.experimental import pallas as pl


def kernel(x, edge_index, eps1, W1a, b1a, W1b, b1b, eps2, W2a, b2a, W2b, b2b, Wh, bh):
    raise NotImplementedError("write your pallas kernel here")


# speedup vs baseline: 4.8195x; 4.8195x over previous
import functools

import jax
import jax.numpy as jnp
from jax import lax
from jax.experimental import pallas as pl
from jax.experimental.pallas import tpu as pltpu
from jax.experimental.pallas import tpu_sc as plsc

N = 10000
E = 320000
NC = 2
NS = 16
NW = NC * NS
EW = E // NW
CH = 80
NCH = EW // CH
RPT = 624
RTAIL = N - NS * RPT


@functools.lru_cache(maxsize=None)
def _make_sc_agg(D: int, via_spmem: bool):
    mesh = plsc.VectorSubcoreMesh(core_axis_name="c", subcore_axis_name="s")
    scratch = [
        pltpu.VMEM((CH,), jnp.int32),
        pltpu.VMEM((CH,), jnp.int32),
        pltpu.VMEM((CH, D), jnp.float32),
        pltpu.VMEM_SHARED((N, D), jnp.float32),
        pltpu.SemaphoreType.DMA,
    ]
    if via_spmem:
        scratch.append(pltpu.VMEM_SHARED((N, D), jnp.float32))

    @functools.partial(
        pl.kernel,
        mesh=mesh,
        out_type=jax.ShapeDtypeStruct((NC, N, D), jnp.float32),
        scratch_types=scratch,
        compiler_params=pltpu.CompilerParams(use_tc_tiling_on_sc=(D == 128)),
    )
    def k(x_hbm, src_hbm, dst_hbm, out_hbm, idx_s, idx_d, rows, acc, sem,
          *maybe_xsp):
        c = lax.axis_index("c")
        s = lax.axis_index("s")
        wid = s * NC + c
        gsrc = maybe_xsp[0] if via_spmem else x_hbm
        pltpu.sync_copy(x_hbm.at[pl.ds(s * RPT, RPT)], acc.at[pl.ds(s * RPT, RPT)])
        if via_spmem:
            pltpu.sync_copy(x_hbm.at[pl.ds(s * RPT, RPT)],
                            gsrc.at[pl.ds(s * RPT, RPT)])

        @pl.when(s == NS - 1)
        def _():
            pltpu.sync_copy(x_hbm.at[pl.ds(NS * RPT, RTAIL)],
                            acc.at[pl.ds(NS * RPT, RTAIL)])
            if via_spmem:
                pltpu.sync_copy(x_hbm.at[pl.ds(NS * RPT, RTAIL)],
                                gsrc.at[pl.ds(NS * RPT, RTAIL)])

        plsc.subcore_barrier()
        base = wid * EW

        def step(i, carry):
            off = base + i * CH
            pltpu.sync_copy(src_hbm.at[pl.ds(off, CH)], idx_s)
            pltpu.sync_copy(dst_hbm.at[pl.ds(off, CH)], idx_d)
            pltpu.async_copy(gsrc.at[idx_s], rows, sem).wait()
            pltpu.sync_copy(rows, acc.at[idx_d], add=True)
            return carry

        lax.fori_loop(0, NCH, step, 0)
        plsc.subcore_barrier()
        pltpu.sync_copy(acc.at[pl.ds(s * RPT, RPT)],
                        out_hbm.at[c, pl.ds(s * RPT, RPT)])

        @pl.when(s == NS - 1)
        def _():
            pltpu.sync_copy(acc.at[pl.ds(NS * RPT, RTAIL)],
                            out_hbm.at[c, pl.ds(NS * RPT, RTAIL)])

    return k


BN = 1000


def _tc1_body(eps_ref, x_ref, agg_ref, W1a_ref, b1a_ref, W1b_ref, b1b_ref,
              W2a_ref, u2_ref):
    z1 = agg_ref[0] + agg_ref[1] + (eps_ref[0, 0] - 1.0) * x_ref[...]
    t = jax.nn.relu(jnp.dot(z1, W1a_ref[...], preferred_element_type=jnp.float32, precision=jax.lax.Precision.HIGHEST)
                    + b1a_ref[...])
    h1 = jax.nn.relu(jnp.dot(t, W1b_ref[...], preferred_element_type=jnp.float32, precision=jax.lax.Precision.HIGHEST)
                     + b1b_ref[...])
    u2_ref[...] = jnp.dot(h1, W2a_ref[...], preferred_element_type=jnp.float32, precision=jax.lax.Precision.HIGHEST)


def _tc1(eps1, x, agg1, W1a, b1a, W1b, b1b, W2a):
    grid = (N // BN,)
    return pl.pallas_call(
        _tc1_body,
        grid=grid,
        in_specs=[
            pl.BlockSpec(memory_space=pltpu.SMEM),
            pl.BlockSpec((BN, 128), lambda i: (i, 0)),
            pl.BlockSpec((NC, BN, 128), lambda i: (0, i, 0)),
            pl.BlockSpec((128, 128), lambda i: (0, 0)),
            pl.BlockSpec((1, 128), lambda i: (0, 0)),
            pl.BlockSpec((128, 128), lambda i: (0, 0)),
            pl.BlockSpec((1, 128), lambda i: (0, 0)),
            pl.BlockSpec((128, 32), lambda i: (0, 0)),
        ],
        out_specs=pl.BlockSpec((BN, 32), lambda i: (i, 0)),
        out_shape=jax.ShapeDtypeStruct((N, 32), jnp.float32),
    )(eps1, x, agg1, W1a, b1a, W1b, b1b, W2a)


def _tc2_body(eps_ref, u2_ref, agg_ref, b2a_ref, W2b_ref, b2b_ref, Wh_ref,
              bh_ref, out_ref):
    z = agg_ref[0] + agg_ref[1] + (eps_ref[0, 0] - 1.0) * u2_ref[...] + b2a_ref[...]
    t = jax.nn.relu(z)
    h2 = jax.nn.relu(jnp.dot(t, W2b_ref[...], preferred_element_type=jnp.float32, precision=jax.lax.Precision.HIGHEST)
                     + b2b_ref[...])
    out_ref[...] = jnp.dot(h2, Wh_ref[...], preferred_element_type=jnp.float32, precision=jax.lax.Precision.HIGHEST) \
        + bh_ref[...]


def _tc2(eps2, u2, agg2, b2a, W2b, b2b, Wh, bh):
    grid = (N // BN,)
    return pl.pallas_call(
        _tc2_body,
        grid=grid,
        in_specs=[
            pl.BlockSpec(memory_space=pltpu.SMEM),
            pl.BlockSpec((BN, 32), lambda i: (i, 0)),
            pl.BlockSpec((NC, BN, 32), lambda i: (0, i, 0)),
            pl.BlockSpec((1, 32), lambda i: (0, 0)),
            pl.BlockSpec((32, 32), lambda i: (0, 0)),
            pl.BlockSpec((1, 32), lambda i: (0, 0)),
            pl.BlockSpec((32, 4), lambda i: (0, 0)),
            pl.BlockSpec((1, 4), lambda i: (0, 0)),
        ],
        out_specs=pl.BlockSpec((BN, 4), lambda i: (i, 0)),
        out_shape=jax.ShapeDtypeStruct((N, 4), jnp.float32),
    )(eps2, u2, agg2, b2a, W2b, b2b, Wh, bh)


def kernel(x, edge_index, eps1, W1a, b1a, W1b, b1b, eps2, W2a, b2a, W2b, b2b,
           Wh, bh):
    src = edge_index[0]
    dst = edge_index[1]
    eps1_s = jnp.reshape(eps1, (1, 1))
    eps2_s = jnp.reshape(eps2, (1, 1))
    agg1 = _make_sc_agg(128, False)(x, src, dst)
    u2 = _tc1(eps1_s, x, agg1,
              W1a, jnp.reshape(b1a, (1, 128)),
              W1b, jnp.reshape(b1b, (1, 128)), W2a)
    agg2 = _make_sc_agg(32, False)(u2, src, dst)
    return _tc2(eps2_s, u2, agg2,
                jnp.reshape(b2a, (1, 32)), W2b, jnp.reshape(b2b, (1, 32)),
                Wh, jnp.reshape(bh, (1, 4)))


# speedup vs baseline: 10.3968x; 2.1572x over previous
import functools

import jax
import jax.numpy as jnp
from jax import lax
from jax.experimental import pallas as pl
from jax.experimental.pallas import tpu as pltpu
from jax.experimental.pallas import tpu_sc as plsc

N = 10000
E = 320000
NC = 2
NS = 16
NW = NC * NS
EW = E // NW
CH = 80
NCH = EW // CH
NPAIR = (NCH - 1) // 2
RPT = N // NS


@functools.lru_cache(maxsize=None)
def _make_sc_agg(D: int):
    mesh = plsc.VectorSubcoreMesh(core_axis_name="c", subcore_axis_name="s")
    scratch = [
        pltpu.VMEM((NCH, CH), jnp.int32),
        pltpu.VMEM((NCH, CH), jnp.int32),
        pltpu.VMEM((CH, D), jnp.float32),
        pltpu.VMEM((CH, D), jnp.float32),
        pltpu.VMEM_SHARED((N, D), jnp.float32),
        pltpu.SemaphoreType.DMA,
        pltpu.SemaphoreType.DMA,
        pltpu.SemaphoreType.DMA,
    ]

    @functools.partial(
        pl.kernel,
        mesh=mesh,
        out_type=jax.ShapeDtypeStruct((NC, N, D), jnp.float32),
        scratch_types=scratch,
        compiler_params=pltpu.CompilerParams(use_tc_tiling_on_sc=False),
    )
    def k(x_hbm, src_hbm, dst_hbm, out_hbm, idxs, idxd, r0, r1, acc,
          sem0, sem1, semi):
        c = lax.axis_index("c")
        s = lax.axis_index("s")
        wid = s * NC + c
        cbase = wid * NCH
        pltpu.async_copy(src_hbm.at[pl.ds(cbase, NCH)], idxs, semi)
        pltpu.async_copy(dst_hbm.at[pl.ds(cbase, NCH)], idxd, semi)
        pltpu.sync_copy(x_hbm.at[pl.ds(s * RPT, RPT)], acc.at[pl.ds(s * RPT, RPT)])
        pltpu.make_async_copy(src_hbm.at[pl.ds(cbase, NCH)], idxs, semi).wait()
        pltpu.make_async_copy(dst_hbm.at[pl.ds(cbase, NCH)], idxd, semi).wait()
        plsc.subcore_barrier()

        def gstart(j, rb, sem):
            pltpu.async_copy(x_hbm.at[idxs.at[j]], rb, sem)

        def gwait(j, rb, sem):
            pltpu.make_async_copy(x_hbm.at[idxs.at[j]], rb, sem).wait()

        def scat(j, rb):
            pltpu.sync_copy(rb, acc.at[idxd.at[j]], add=True)

        gstart(0, r0, sem0)

        def pair(j, carry):
            a = 2 * j
            b = a + 1
            gstart(b, r1, sem1)
            gwait(a, r0, sem0)
            scat(a, r0)
            gstart(a + 2, r0, sem0)
            gwait(b, r1, sem1)
            scat(b, r1)
            return carry

        lax.fori_loop(0, NPAIR, pair, 0)
        gwait(NCH - 1, r0, sem0)
        scat(NCH - 1, r0)
        plsc.subcore_barrier()
        pltpu.sync_copy(acc.at[pl.ds(s * RPT, RPT)],
                        out_hbm.at[c, pl.ds(s * RPT, RPT)])

    return k


BN = 1000


def _tc1_body(eps_ref, x_ref, agg_ref, W1a_ref, b1a_ref, W1b_ref, b1b_ref,
              W2a_ref, u2_ref):
    z1 = agg_ref[0] + agg_ref[1] + (eps_ref[0, 0] - 1.0) * x_ref[...]
    t = jax.nn.relu(jnp.dot(z1, W1a_ref[...], preferred_element_type=jnp.float32, precision=jax.lax.Precision.HIGHEST)
                    + b1a_ref[...])
    h1 = jax.nn.relu(jnp.dot(t, W1b_ref[...], preferred_element_type=jnp.float32, precision=jax.lax.Precision.HIGHEST)
                     + b1b_ref[...])
    u2_ref[...] = jnp.dot(h1, W2a_ref[...], preferred_element_type=jnp.float32, precision=jax.lax.Precision.HIGHEST)


def _tc1(eps1, x, agg1, W1a, b1a, W1b, b1b, W2a):
    grid = (N // BN,)
    return pl.pallas_call(
        _tc1_body,
        grid=grid,
        in_specs=[
            pl.BlockSpec(memory_space=pltpu.SMEM),
            pl.BlockSpec((BN, 128), lambda i: (i, 0)),
            pl.BlockSpec((NC, BN, 128), lambda i: (0, i, 0)),
            pl.BlockSpec((128, 128), lambda i: (0, 0)),
            pl.BlockSpec((1, 128), lambda i: (0, 0)),
            pl.BlockSpec((128, 128), lambda i: (0, 0)),
            pl.BlockSpec((1, 128), lambda i: (0, 0)),
            pl.BlockSpec((128, 32), lambda i: (0, 0)),
        ],
        out_specs=pl.BlockSpec((BN, 32), lambda i: (i, 0)),
        out_shape=jax.ShapeDtypeStruct((N, 32), jnp.float32),
    )(eps1, x, agg1, W1a, b1a, W1b, b1b, W2a)


def _tc2_body(eps_ref, u2_ref, agg_ref, b2a_ref, W2b_ref, b2b_ref, Wh_ref,
              bh_ref, out_ref):
    z = agg_ref[0] + agg_ref[1] + (eps_ref[0, 0] - 1.0) * u2_ref[...] + b2a_ref[...]
    t = jax.nn.relu(z)
    h2 = jax.nn.relu(jnp.dot(t, W2b_ref[...], preferred_element_type=jnp.float32, precision=jax.lax.Precision.HIGHEST)
                     + b2b_ref[...])
    out_ref[...] = jnp.dot(h2, Wh_ref[...], preferred_element_type=jnp.float32, precision=jax.lax.Precision.HIGHEST) \
        + bh_ref[...]


def _tc2(eps2, u2, agg2, b2a, W2b, b2b, Wh, bh):
    grid = (N // BN,)
    return pl.pallas_call(
        _tc2_body,
        grid=grid,
        in_specs=[
            pl.BlockSpec(memory_space=pltpu.SMEM),
            pl.BlockSpec((BN, 32), lambda i: (i, 0)),
            pl.BlockSpec((NC, BN, 32), lambda i: (0, i, 0)),
            pl.BlockSpec((1, 32), lambda i: (0, 0)),
            pl.BlockSpec((32, 32), lambda i: (0, 0)),
            pl.BlockSpec((1, 32), lambda i: (0, 0)),
            pl.BlockSpec((32, 4), lambda i: (0, 0)),
            pl.BlockSpec((1, 4), lambda i: (0, 0)),
        ],
        out_specs=pl.BlockSpec((BN, 4), lambda i: (i, 0)),
        out_shape=jax.ShapeDtypeStruct((N, 4), jnp.float32),
    )(eps2, u2, agg2, b2a, W2b, b2b, Wh, bh)


def kernel(x, edge_index, eps1, W1a, b1a, W1b, b1b, eps2, W2a, b2a, W2b, b2b,
           Wh, bh):
    src2 = jnp.reshape(edge_index[0], (E // CH, CH))
    dst2 = jnp.reshape(edge_index[1], (E // CH, CH))
    eps1_s = jnp.reshape(eps1, (1, 1))
    eps2_s = jnp.reshape(eps2, (1, 1))
    agg1 = _make_sc_agg(128)(x, src2, dst2)
    u2 = _tc1(eps1_s, x, agg1,
              W1a, jnp.reshape(b1a, (1, 128)),
              W1b, jnp.reshape(b1b, (1, 128)), W2a)
    agg2 = _make_sc_agg(32)(u2, src2, dst2)
    return _tc2(eps2_s, u2, agg2,
                jnp.reshape(b2a, (1, 32)), W2b, jnp.reshape(b2b, (1, 32)),
                Wh, jnp.reshape(bh, (1, 4)))


# speedup vs baseline: 11.3990x; 1.0964x over previous
import functools

import jax
import jax.numpy as jnp
from jax import lax
from jax.experimental import pallas as pl
from jax.experimental.pallas import tpu as pltpu
from jax.experimental.pallas import tpu_sc as plsc

N = 10000
E = 320000
NC = 2
NS = 16
NW = NC * NS
EW = E // NW
CH = 80
NCH = EW // CH
NPAIR = (NCH - 1) // 2
RPT = N // NS


@functools.lru_cache(maxsize=None)
def _make_sc_agg(D: int):
    mesh = plsc.VectorSubcoreMesh(core_axis_name="c", subcore_axis_name="s")
    NB = 3 if D == 128 else 8
    scratch = [
        pltpu.VMEM((NCH, CH), jnp.int32),
        pltpu.VMEM((NCH, CH), jnp.int32),
        [pltpu.VMEM((CH, D), jnp.float32) for _ in range(NB)],
        pltpu.VMEM_SHARED((N, D), jnp.float32),
        [pltpu.SemaphoreType.DMA for _ in range(NB)],
        pltpu.SemaphoreType.DMA,
    ]

    @functools.partial(
        pl.kernel,
        mesh=mesh,
        out_type=jax.ShapeDtypeStruct((NC, N, D), jnp.float32),
        scratch_types=scratch,
        compiler_params=pltpu.CompilerParams(use_tc_tiling_on_sc=False),
    )
    def k(x_hbm, src_hbm, dst_hbm, out_hbm, idxs, idxd, bufs, acc, sems, semi):
        c = lax.axis_index("c")
        s = lax.axis_index("s")
        wid = s * NC + c
        cbase = wid * NCH
        pltpu.async_copy(src_hbm.at[pl.ds(cbase, NCH)], idxs, semi)
        pltpu.async_copy(dst_hbm.at[pl.ds(cbase, NCH)], idxd, semi)
        pltpu.sync_copy(x_hbm.at[pl.ds(s * RPT, RPT)], acc.at[pl.ds(s * RPT, RPT)])
        pltpu.make_async_copy(src_hbm.at[pl.ds(cbase, NCH)], idxs, semi).wait()
        pltpu.make_async_copy(dst_hbm.at[pl.ds(cbase, NCH)], idxd, semi).wait()
        plsc.subcore_barrier()

        def gstart(j, rb, sem):
            pltpu.async_copy(x_hbm.at[idxs.at[j]], rb, sem)

        def gwait(j, rb, sem):
            pltpu.make_async_copy(x_hbm.at[idxs.at[j]], rb, sem).wait()

        def sstart(j, rb, sem):
            pltpu.async_copy(rb, acc.at[idxd.at[j]], sem, add=True)

        def swait(j, rb, sem):
            pltpu.make_async_copy(rb, acc.at[idxd.at[j]], sem).wait()

        NRND = NCH // NB
        TAIL = NCH % NB
        for t in range(NB):
            gstart(t, bufs[t], sems[t])

        def rnd(q, carry):
            jb = q * NB
            for t in range(NB):
                gwait(jb + t, bufs[t], sems[t])
                sstart(jb + t, bufs[t], sems[t])
            for t in range(NB):
                swait(jb + t, bufs[t], sems[t])
                gstart(jb + NB + t, bufs[t], sems[t])
            return carry

        lax.fori_loop(0, NRND - 1, rnd, 0)
        jb = (NRND - 1) * NB
        for t in range(NB):
            gwait(jb + t, bufs[t], sems[t])
            sstart(jb + t, bufs[t], sems[t])
        for t in range(TAIL):
            swait(jb + t, bufs[t], sems[t])
            gstart(jb + NB + t, bufs[t], sems[t])
        for t in range(TAIL):
            gwait(jb + NB + t, bufs[t], sems[t])
            sstart(jb + NB + t, bufs[t], sems[t])
        for t in range(TAIL):
            swait(jb + NB + t, bufs[t], sems[t])
        for t in range(TAIL, NB):
            swait(jb + t, bufs[t], sems[t])
        plsc.subcore_barrier()
        pltpu.sync_copy(acc.at[pl.ds(s * RPT, RPT)],
                        out_hbm.at[c, pl.ds(s * RPT, RPT)])

    return k


BN = 1000


def _tc1_body(eps_ref, x_ref, agg_ref, W1a_ref, b1a_ref, W1b_ref, b1b_ref,
              W2a_ref, u2_ref):
    z1 = agg_ref[0] + agg_ref[1] + (eps_ref[0, 0] - 1.0) * x_ref[...]
    t = jax.nn.relu(jnp.dot(z1, W1a_ref[...], preferred_element_type=jnp.float32, precision=jax.lax.Precision.HIGHEST)
                    + b1a_ref[...])
    h1 = jax.nn.relu(jnp.dot(t, W1b_ref[...], preferred_element_type=jnp.float32, precision=jax.lax.Precision.HIGHEST)
                     + b1b_ref[...])
    u2_ref[...] = jnp.dot(h1, W2a_ref[...], preferred_element_type=jnp.float32, precision=jax.lax.Precision.HIGHEST)


def _tc1(eps1, x, agg1, W1a, b1a, W1b, b1b, W2a):
    grid = (N // BN,)
    return pl.pallas_call(
        _tc1_body,
        grid=grid,
        in_specs=[
            pl.BlockSpec(memory_space=pltpu.SMEM),
            pl.BlockSpec((BN, 128), lambda i: (i, 0)),
            pl.BlockSpec((NC, BN, 128), lambda i: (0, i, 0)),
            pl.BlockSpec((128, 128), lambda i: (0, 0)),
            pl.BlockSpec((1, 128), lambda i: (0, 0)),
            pl.BlockSpec((128, 128), lambda i: (0, 0)),
            pl.BlockSpec((1, 128), lambda i: (0, 0)),
            pl.BlockSpec((128, 32), lambda i: (0, 0)),
        ],
        out_specs=pl.BlockSpec((BN, 32), lambda i: (i, 0)),
        out_shape=jax.ShapeDtypeStruct((N, 32), jnp.float32),
    )(eps1, x, agg1, W1a, b1a, W1b, b1b, W2a)


def _tc2_body(eps_ref, u2_ref, agg_ref, b2a_ref, W2b_ref, b2b_ref, Wh_ref,
              bh_ref, out_ref):
    z = agg_ref[0] + agg_ref[1] + (eps_ref[0, 0] - 1.0) * u2_ref[...] + b2a_ref[...]
    t = jax.nn.relu(z)
    h2 = jax.nn.relu(jnp.dot(t, W2b_ref[...], preferred_element_type=jnp.float32, precision=jax.lax.Precision.HIGHEST)
                     + b2b_ref[...])
    out_ref[...] = jnp.dot(h2, Wh_ref[...], preferred_element_type=jnp.float32, precision=jax.lax.Precision.HIGHEST) \
        + bh_ref[...]


def _tc2(eps2, u2, agg2, b2a, W2b, b2b, Wh, bh):
    grid = (N // BN,)
    return pl.pallas_call(
        _tc2_body,
        grid=grid,
        in_specs=[
            pl.BlockSpec(memory_space=pltpu.SMEM),
            pl.BlockSpec((BN, 32), lambda i: (i, 0)),
            pl.BlockSpec((NC, BN, 32), lambda i: (0, i, 0)),
            pl.BlockSpec((1, 32), lambda i: (0, 0)),
            pl.BlockSpec((32, 32), lambda i: (0, 0)),
            pl.BlockSpec((1, 32), lambda i: (0, 0)),
            pl.BlockSpec((32, 4), lambda i: (0, 0)),
            pl.BlockSpec((1, 4), lambda i: (0, 0)),
        ],
        out_specs=pl.BlockSpec((BN, 4), lambda i: (i, 0)),
        out_shape=jax.ShapeDtypeStruct((N, 4), jnp.float32),
    )(eps2, u2, agg2, b2a, W2b, b2b, Wh, bh)


def kernel(x, edge_index, eps1, W1a, b1a, W1b, b1b, eps2, W2a, b2a, W2b, b2b,
           Wh, bh):
    src2 = jnp.reshape(edge_index[0], (E // CH, CH))
    dst2 = jnp.reshape(edge_index[1], (E // CH, CH))
    eps1_s = jnp.reshape(eps1, (1, 1))
    eps2_s = jnp.reshape(eps2, (1, 1))
    agg1 = _make_sc_agg(128)(x, src2, dst2)
    u2 = _tc1(eps1_s, x, agg1,
              W1a, jnp.reshape(b1a, (1, 128)),
              W1b, jnp.reshape(b1b, (1, 128)), W2a)
    agg2 = _make_sc_agg(32)(u2, src2, dst2)
    return _tc2(eps2_s, u2, agg2,
                jnp.reshape(b2a, (1, 32)), W2b, jnp.reshape(b2b, (1, 32)),
                Wh, jnp.reshape(bh, (1, 4)))


# speedup vs baseline: 13.3772x; 1.1735x over previous
import functools

import jax
import jax.numpy as jnp
from jax import lax
from jax.experimental import pallas as pl
from jax.experimental.pallas import tpu as pltpu
from jax.experimental.pallas import tpu_sc as plsc

N = 10000
E = 320000
NC = 2
NS = 16
NW = NC * NS
EW = E // NW
CH = 80
NCH = EW // CH
NPAIR = (NCH - 1) // 2
RPT = N // NS


@functools.lru_cache(maxsize=None)
def _make_sc_agg(D: int):
    mesh = plsc.VectorSubcoreMesh(core_axis_name="c", subcore_axis_name="s")
    NB = 3 if D == 128 else 8
    scratch = [
        pltpu.VMEM((EW,), jnp.int32),
        pltpu.VMEM((EW,), jnp.int32),
        [pltpu.VMEM((CH, D), jnp.float32) for _ in range(NB)],
        pltpu.VMEM_SHARED((N, D), jnp.float32),
        [pltpu.SemaphoreType.DMA for _ in range(NB)],
        pltpu.SemaphoreType.DMA,
    ]

    @functools.partial(
        pl.kernel,
        mesh=mesh,
        out_type=jax.ShapeDtypeStruct((NC, N, D), jnp.float32),
        scratch_types=scratch,
        compiler_params=pltpu.CompilerParams(use_tc_tiling_on_sc=False),
    )
    def k(x_hbm, ei_hbm, out_hbm, idxs, idxd, bufs, acc, sems, semi):
        c = lax.axis_index("c")
        s = lax.axis_index("s")
        wid = s * NC + c
        ebase = wid * EW
        pltpu.async_copy(ei_hbm.at[pl.ds(ebase, EW)], idxs, semi)
        pltpu.async_copy(ei_hbm.at[pl.ds(E + ebase, EW)], idxd, semi)
        pltpu.sync_copy(x_hbm.at[pl.ds(s * RPT, RPT)], acc.at[pl.ds(s * RPT, RPT)])
        pltpu.make_async_copy(ei_hbm.at[pl.ds(ebase, EW)], idxs, semi).wait()
        pltpu.make_async_copy(ei_hbm.at[pl.ds(E + ebase, EW)], idxd, semi).wait()
        plsc.subcore_barrier()

        def gstart(j, rb, sem):
            pltpu.async_copy(x_hbm.at[idxs.at[pl.ds(j * CH, CH)]], rb, sem)

        def gwait(j, rb, sem):
            pltpu.make_async_copy(x_hbm.at[idxs.at[pl.ds(j * CH, CH)]], rb,
                                  sem).wait()

        def sstart(j, rb, sem):
            pltpu.async_copy(rb, acc.at[idxd.at[pl.ds(j * CH, CH)]], sem,
                             add=True)

        def swait(j, rb, sem):
            pltpu.make_async_copy(rb, acc.at[idxd.at[pl.ds(j * CH, CH)]],
                                  sem).wait()

        NRND = NCH // NB
        TAIL = NCH % NB
        for t in range(NB):
            gstart(t, bufs[t], sems[t])

        def rnd(q, carry):
            jb = q * NB
            for t in range(NB):
                gwait(jb + t, bufs[t], sems[t])
                sstart(jb + t, bufs[t], sems[t])
            for t in range(NB):
                swait(jb + t, bufs[t], sems[t])
                gstart(jb + NB + t, bufs[t], sems[t])
            return carry

        lax.fori_loop(0, NRND - 1, rnd, 0)
        jb = (NRND - 1) * NB
        for t in range(NB):
            gwait(jb + t, bufs[t], sems[t])
            sstart(jb + t, bufs[t], sems[t])
        for t in range(TAIL):
            swait(jb + t, bufs[t], sems[t])
            gstart(jb + NB + t, bufs[t], sems[t])
        for t in range(TAIL):
            gwait(jb + NB + t, bufs[t], sems[t])
            sstart(jb + NB + t, bufs[t], sems[t])
        for t in range(TAIL):
            swait(jb + NB + t, bufs[t], sems[t])
        for t in range(TAIL, NB):
            swait(jb + t, bufs[t], sems[t])
        plsc.subcore_barrier()
        pltpu.sync_copy(acc.at[pl.ds(s * RPT, RPT)],
                        out_hbm.at[c, pl.ds(s * RPT, RPT)])

    return k


BN = 2000
BN2 = 10000


def _tc1_body(eps_ref, x_ref, agg_ref, W1a_ref, b1a_ref, W1b_ref, b1b_ref,
              W2a_ref, u2_ref):
    z1 = agg_ref[0] + agg_ref[1] + (eps_ref[0, 0] - 1.0) * x_ref[...]
    t = jax.nn.relu(jnp.dot(z1, W1a_ref[...], preferred_element_type=jnp.float32, precision=jax.lax.Precision.HIGHEST)
                    + b1a_ref[...])
    h1 = jax.nn.relu(jnp.dot(t, W1b_ref[...], preferred_element_type=jnp.float32, precision=jax.lax.Precision.HIGHEST)
                     + b1b_ref[...])
    u2_ref[...] = jnp.dot(h1, W2a_ref[...], preferred_element_type=jnp.float32, precision=jax.lax.Precision.HIGHEST)


def _tc1(eps1, x, agg1, W1a, b1a, W1b, b1b, W2a):
    grid = (N // BN,)
    return pl.pallas_call(
        _tc1_body,
        grid=grid,
        in_specs=[
            pl.BlockSpec(memory_space=pltpu.SMEM),
            pl.BlockSpec((BN, 128), lambda i: (i, 0)),
            pl.BlockSpec((NC, BN, 128), lambda i: (0, i, 0)),
            pl.BlockSpec((128, 128), lambda i: (0, 0)),
            pl.BlockSpec((1, 128), lambda i: (0, 0)),
            pl.BlockSpec((128, 128), lambda i: (0, 0)),
            pl.BlockSpec((1, 128), lambda i: (0, 0)),
            pl.BlockSpec((128, 32), lambda i: (0, 0)),
        ],
        out_specs=pl.BlockSpec((BN, 32), lambda i: (i, 0)),
        out_shape=jax.ShapeDtypeStruct((N, 32), jnp.float32),
    )(eps1, x, agg1, W1a, b1a, W1b, b1b, W2a)


def _tc2_body(eps_ref, u2_ref, agg_ref, b2a_ref, W2b_ref, b2b_ref, Wh_ref,
              bh_ref, out_ref):
    z = agg_ref[0] + agg_ref[1] + (eps_ref[0, 0] - 1.0) * u2_ref[...] + b2a_ref[...]
    t = jax.nn.relu(z)
    h2 = jax.nn.relu(jnp.dot(t, W2b_ref[...], preferred_element_type=jnp.float32, precision=jax.lax.Precision.HIGHEST)
                     + b2b_ref[...])
    out_ref[...] = jnp.dot(h2, Wh_ref[...], preferred_element_type=jnp.float32, precision=jax.lax.Precision.HIGHEST) \
        + bh_ref[...]


def _tc2(eps2, u2, agg2, b2a, W2b, b2b, Wh, bh):
    grid = (N // BN2,)
    return pl.pallas_call(
        _tc2_body,
        grid=grid,
        in_specs=[
            pl.BlockSpec(memory_space=pltpu.SMEM),
            pl.BlockSpec((BN2, 32), lambda i: (i, 0)),
            pl.BlockSpec((NC, BN2, 32), lambda i: (0, i, 0)),
            pl.BlockSpec((1, 32), lambda i: (0, 0)),
            pl.BlockSpec((32, 32), lambda i: (0, 0)),
            pl.BlockSpec((1, 32), lambda i: (0, 0)),
            pl.BlockSpec((32, 4), lambda i: (0, 0)),
            pl.BlockSpec((1, 4), lambda i: (0, 0)),
        ],
        out_specs=pl.BlockSpec((BN2, 4), lambda i: (i, 0)),
        out_shape=jax.ShapeDtypeStruct((N, 4), jnp.float32),
    )(eps2, u2, agg2, b2a, W2b, b2b, Wh, bh)


def kernel(x, edge_index, eps1, W1a, b1a, W1b, b1b, eps2, W2a, b2a, W2b, b2b,
           Wh, bh):
    ei = jnp.reshape(edge_index, (2 * E,))
    eps1_s = jnp.reshape(eps1, (1, 1))
    eps2_s = jnp.reshape(eps2, (1, 1))
    agg1 = _make_sc_agg(128)(x, ei)
    u2 = _tc1(eps1_s, x, agg1,
              W1a, jnp.reshape(b1a, (1, 128)),
              W1b, jnp.reshape(b1b, (1, 128)), W2a)
    agg2 = _make_sc_agg(32)(u2, ei)
    return _tc2(eps2_s, u2, agg2,
                jnp.reshape(b2a, (1, 32)), W2b, jnp.reshape(b2b, (1, 32)),
                Wh, jnp.reshape(bh, (1, 4)))


# speedup vs baseline: 13.9329x; 1.0415x over previous
import functools

import jax
import jax.numpy as jnp
from jax import lax
from jax.experimental import pallas as pl
from jax.experimental.pallas import tpu as pltpu
from jax.experimental.pallas import tpu_sc as plsc

N = 10000
E = 320000
NC = 2
NS = 16
NW = NC * NS
EW = E // NW
RPT = N // NS


@functools.lru_cache(maxsize=None)
def _make_sc_agg(D: int):
    mesh = plsc.VectorSubcoreMesh(core_axis_name="c", subcore_axis_name="s")
    CH = 40 if D == 128 else 80
    NCH = EW // CH
    NB = 6 if D == 128 else 8
    scratch = [
        pltpu.VMEM((EW,), jnp.int32),
        pltpu.VMEM((EW,), jnp.int32),
        [pltpu.VMEM((CH, D), jnp.float32) for _ in range(NB)],
        pltpu.VMEM_SHARED((N, D), jnp.float32),
        [pltpu.SemaphoreType.DMA for _ in range(NB)],
        pltpu.SemaphoreType.DMA,
    ]

    @functools.partial(
        pl.kernel,
        mesh=mesh,
        out_type=jax.ShapeDtypeStruct((NC, N, D), jnp.float32),
        scratch_types=scratch,
        compiler_params=pltpu.CompilerParams(use_tc_tiling_on_sc=False),
    )
    def k(x_hbm, ei_hbm, out_hbm, idxs, idxd, bufs, acc, sems, semi):
        c = lax.axis_index("c")
        s = lax.axis_index("s")
        wid = s * NC + c
        ebase = wid * EW
        pltpu.async_copy(ei_hbm.at[pl.ds(ebase, EW)], idxs, semi)
        pltpu.async_copy(ei_hbm.at[pl.ds(E + ebase, EW)], idxd, semi)
        pltpu.sync_copy(x_hbm.at[pl.ds(s * RPT, RPT)], acc.at[pl.ds(s * RPT, RPT)])
        pltpu.make_async_copy(ei_hbm.at[pl.ds(ebase, EW)], idxs, semi).wait()
        pltpu.make_async_copy(ei_hbm.at[pl.ds(E + ebase, EW)], idxd, semi).wait()
        plsc.subcore_barrier()

        def gstart(j, rb, sem):
            pltpu.async_copy(x_hbm.at[idxs.at[pl.ds(j * CH, CH)]], rb, sem)

        def gwait(j, rb, sem):
            pltpu.make_async_copy(x_hbm.at[idxs.at[pl.ds(j * CH, CH)]], rb,
                                  sem).wait()

        def sstart(j, rb, sem):
            pltpu.async_copy(rb, acc.at[idxd.at[pl.ds(j * CH, CH)]], sem,
                             add=True)

        def swait(j, rb, sem):
            pltpu.make_async_copy(rb, acc.at[idxd.at[pl.ds(j * CH, CH)]],
                                  sem).wait()

        NRND = NCH // NB
        TAIL = NCH % NB
        for t in range(NB):
            gstart(t, bufs[t], sems[t])

        def rnd(q, carry):
            jb = q * NB
            for t in range(NB):
                gwait(jb + t, bufs[t], sems[t])
                sstart(jb + t, bufs[t], sems[t])
            for t in range(NB):
                swait(jb + t, bufs[t], sems[t])
                gstart(jb + NB + t, bufs[t], sems[t])
            return carry

        lax.fori_loop(0, NRND - 1, rnd, 0)
        jb = (NRND - 1) * NB
        for t in range(NB):
            gwait(jb + t, bufs[t], sems[t])
            sstart(jb + t, bufs[t], sems[t])
        for t in range(TAIL):
            swait(jb + t, bufs[t], sems[t])
            gstart(jb + NB + t, bufs[t], sems[t])
        for t in range(TAIL):
            gwait(jb + NB + t, bufs[t], sems[t])
            sstart(jb + NB + t, bufs[t], sems[t])
        for t in range(TAIL):
            swait(jb + NB + t, bufs[t], sems[t])
        for t in range(TAIL, NB):
            swait(jb + t, bufs[t], sems[t])
        plsc.subcore_barrier()
        pltpu.sync_copy(acc.at[pl.ds(s * RPT, RPT)],
                        out_hbm.at[c, pl.ds(s * RPT, RPT)])

    return k


BN = 2000
BN2 = 10000


def _tc1_body(eps_ref, x_ref, agg_ref, W1a_ref, b1a_ref, W1b_ref, b1b_ref,
              W2a_ref, u2_ref):
    z1 = agg_ref[0] + agg_ref[1] + (eps_ref[0, 0] - 1.0) * x_ref[...]
    t = jax.nn.relu(jnp.dot(z1, W1a_ref[...], preferred_element_type=jnp.float32, precision=jax.lax.Precision.HIGHEST)
                    + b1a_ref[...])
    h1 = jax.nn.relu(jnp.dot(t, W1b_ref[...], preferred_element_type=jnp.float32, precision=jax.lax.Precision.HIGHEST)
                     + b1b_ref[...])
    u2_ref[...] = jnp.dot(h1, W2a_ref[...], preferred_element_type=jnp.float32, precision=jax.lax.Precision.HIGHEST)


def _tc1(eps1, x, agg1, W1a, b1a, W1b, b1b, W2a):
    grid = (N // BN,)
    return pl.pallas_call(
        _tc1_body,
        grid=grid,
        in_specs=[
            pl.BlockSpec(memory_space=pltpu.SMEM),
            pl.BlockSpec((BN, 128), lambda i: (i, 0)),
            pl.BlockSpec((NC, BN, 128), lambda i: (0, i, 0)),
            pl.BlockSpec((128, 128), lambda i: (0, 0)),
            pl.BlockSpec((1, 128), lambda i: (0, 0)),
            pl.BlockSpec((128, 128), lambda i: (0, 0)),
            pl.BlockSpec((1, 128), lambda i: (0, 0)),
            pl.BlockSpec((128, 32), lambda i: (0, 0)),
        ],
        out_specs=pl.BlockSpec((BN, 32), lambda i: (i, 0)),
        out_shape=jax.ShapeDtypeStruct((N, 32), jnp.float32),
    )(eps1, x, agg1, W1a, b1a, W1b, b1b, W2a)


def _tc2_body(eps_ref, u2_ref, agg_ref, b2a_ref, W2b_ref, b2b_ref, Wh_ref,
              bh_ref, out_ref):
    z = agg_ref[0] + agg_ref[1] + (eps_ref[0, 0] - 1.0) * u2_ref[...] + b2a_ref[...]
    t = jax.nn.relu(z)
    h2 = jax.nn.relu(jnp.dot(t, W2b_ref[...], preferred_element_type=jnp.float32, precision=jax.lax.Precision.HIGHEST)
                     + b2b_ref[...])
    out_ref[...] = jnp.dot(h2, Wh_ref[...], preferred_element_type=jnp.float32, precision=jax.lax.Precision.HIGHEST) \
        + bh_ref[...]


def _tc2(eps2, u2, agg2, b2a, W2b, b2b, Wh, bh):
    grid = (N // BN2,)
    return pl.pallas_call(
        _tc2_body,
        grid=grid,
        in_specs=[
            pl.BlockSpec(memory_space=pltpu.SMEM),
            pl.BlockSpec((BN2, 32), lambda i: (i, 0)),
            pl.BlockSpec((NC, BN2, 32), lambda i: (0, i, 0)),
            pl.BlockSpec((1, 32), lambda i: (0, 0)),
            pl.BlockSpec((32, 32), lambda i: (0, 0)),
            pl.BlockSpec((1, 32), lambda i: (0, 0)),
            pl.BlockSpec((32, 4), lambda i: (0, 0)),
            pl.BlockSpec((1, 4), lambda i: (0, 0)),
        ],
        out_specs=pl.BlockSpec((BN2, 4), lambda i: (i, 0)),
        out_shape=jax.ShapeDtypeStruct((N, 4), jnp.float32),
    )(eps2, u2, agg2, b2a, W2b, b2b, Wh, bh)


def kernel(x, edge_index, eps1, W1a, b1a, W1b, b1b, eps2, W2a, b2a, W2b, b2b,
           Wh, bh):
    ei = jnp.reshape(edge_index, (2 * E,))
    eps1_s = jnp.reshape(eps1, (1, 1))
    eps2_s = jnp.reshape(eps2, (1, 1))
    agg1 = _make_sc_agg(128)(x, ei)
    u2 = _tc1(eps1_s, x, agg1,
              W1a, jnp.reshape(b1a, (1, 128)),
              W1b, jnp.reshape(b1b, (1, 128)), W2a)
    agg2 = _make_sc_agg(32)(u2, ei)
    return _tc2(eps2_s, u2, agg2,
                jnp.reshape(b2a, (1, 32)), W2b, jnp.reshape(b2b, (1, 32)),
                Wh, jnp.reshape(bh, (1, 4)))


# speedup vs baseline: 14.8927x; 1.0689x over previous
import functools

import jax
import jax.numpy as jnp
from jax import lax
from jax.experimental import pallas as pl
from jax.experimental.pallas import tpu as pltpu
from jax.experimental.pallas import tpu_sc as plsc

N = 10000
E = 320000
NC = 2
NS = 16
NW = NC * NS
EW = E // NW
RPT = N // NS


@functools.lru_cache(maxsize=None)
def _make_sc_agg(D: int):
    mesh = plsc.VectorSubcoreMesh(core_axis_name="c", subcore_axis_name="s")
    CH = 40 if D == 128 else 80
    NCH = EW // CH
    NB = 6 if D == 128 else 8
    scratch = [
        pltpu.VMEM((EW,), jnp.int32),
        pltpu.VMEM((EW,), jnp.int32),
        [pltpu.VMEM((CH, D), jnp.float32) for _ in range(NB)],
        pltpu.VMEM_SHARED((N, D), jnp.float32),
        [pltpu.SemaphoreType.DMA for _ in range(NB)],
        pltpu.SemaphoreType.DMA,
    ]

    @functools.partial(
        pl.kernel,
        mesh=mesh,
        out_type=jax.ShapeDtypeStruct((NC, N, D), jnp.float32),
        scratch_types=scratch,
        compiler_params=pltpu.CompilerParams(use_tc_tiling_on_sc=False),
    )
    def k(x_hbm, ei_hbm, out_hbm, idxs, idxd, bufs, acc, sems, semi):
        c = lax.axis_index("c")
        s = lax.axis_index("s")
        wid = s * NC + c
        ebase = wid * EW
        pltpu.async_copy(ei_hbm.at[pl.ds(ebase, EW)], idxs, semi)
        pltpu.async_copy(ei_hbm.at[pl.ds(E + ebase, EW)], idxd, semi)
        pltpu.sync_copy(x_hbm.at[pl.ds(s * RPT, RPT)], acc.at[pl.ds(s * RPT, RPT)])
        pltpu.make_async_copy(ei_hbm.at[pl.ds(ebase, EW)], idxs, semi).wait()
        pltpu.make_async_copy(ei_hbm.at[pl.ds(E + ebase, EW)], idxd, semi).wait()
        plsc.subcore_barrier()

        def gstart(j, rb, sem):
            pltpu.async_copy(x_hbm.at[idxs.at[pl.ds(j * CH, CH)]], rb, sem)

        def gwait(j, rb, sem):
            pltpu.make_async_copy(x_hbm.at[idxs.at[pl.ds(j * CH, CH)]], rb,
                                  sem).wait()

        def sstart(j, rb, sem):
            pltpu.async_copy(rb, acc.at[idxd.at[pl.ds(j * CH, CH)]], sem,
                             add=True)

        def swait(j, rb, sem):
            pltpu.make_async_copy(rb, acc.at[idxd.at[pl.ds(j * CH, CH)]],
                                  sem).wait()

        NRND = NCH // NB
        TAIL = NCH % NB
        for t in range(NB):
            gstart(t, bufs[t], sems[t])

        def rnd(q, carry):
            jb = q * NB
            for t in range(NB):
                gwait(jb + t, bufs[t], sems[t])
                sstart(jb + t, bufs[t], sems[t])
            for t in range(NB):
                swait(jb + t, bufs[t], sems[t])
                gstart(jb + NB + t, bufs[t], sems[t])
            return carry

        lax.fori_loop(0, NRND - 1, rnd, 0)
        jb = (NRND - 1) * NB
        for t in range(NB):
            gwait(jb + t, bufs[t], sems[t])
            sstart(jb + t, bufs[t], sems[t])
        for t in range(TAIL):
            swait(jb + t, bufs[t], sems[t])
            gstart(jb + NB + t, bufs[t], sems[t])
        for t in range(TAIL):
            gwait(jb + NB + t, bufs[t], sems[t])
            sstart(jb + NB + t, bufs[t], sems[t])
        for t in range(TAIL):
            swait(jb + NB + t, bufs[t], sems[t])
        for t in range(TAIL, NB):
            swait(jb + t, bufs[t], sems[t])
        plsc.subcore_barrier()
        pltpu.sync_copy(acc.at[pl.ds(s * RPT, RPT)],
                        out_hbm.at[c, pl.ds(s * RPT, RPT)])

    return k


BN = 2000


N4 = N // 4


def _tc1_body(eps_ref, x_ref, agg_ref, W1a_ref, b1a_ref, W1b_ref, b1b_ref,
              W2a_ref, u2_ref):
    z1 = agg_ref[0] + agg_ref[1] + (eps_ref[0, 0] - 1.0) * x_ref[...]
    t = jax.nn.relu(jnp.dot(z1, W1a_ref[...], preferred_element_type=jnp.float32, precision=jax.lax.Precision.HIGHEST)
                    + b1a_ref[...])
    h1 = jax.nn.relu(jnp.dot(t, W1b_ref[...], preferred_element_type=jnp.float32, precision=jax.lax.Precision.HIGHEST)
                     + b1b_ref[...])
    u2_ref[...] = jnp.dot(h1, W2a_ref[...], preferred_element_type=jnp.float32, precision=jax.lax.Precision.HIGHEST)


def _tc1(eps1, x, agg1, W1a, b1a, W1b, b1b, W2a):
    grid = (N // BN,)
    return pl.pallas_call(
        _tc1_body,
        grid=grid,
        in_specs=[
            pl.BlockSpec(memory_space=pltpu.SMEM),
            pl.BlockSpec((BN, 128), lambda i: (i, 0)),
            pl.BlockSpec((NC, BN, 128), lambda i: (0, i, 0)),
            pl.BlockSpec((128, 128), lambda i: (0, 0)),
            pl.BlockSpec((1, 128), lambda i: (0, 0)),
            pl.BlockSpec((128, 128), lambda i: (0, 0)),
            pl.BlockSpec((1, 128), lambda i: (0, 0)),
            pl.BlockSpec((128, 32), lambda i: (0, 0)),
        ],
        out_specs=pl.BlockSpec((BN, 32), lambda i: (i, 0)),
        out_shape=jax.ShapeDtypeStruct((N, 32), jnp.float32),
    )(eps1, x, agg1, W1a, b1a, W1b, b1b, W2a)


def _tc2_body(eps_ref, u2_ref, agg_ref, b2a_ref, W2b_ref, b2b_ref, Wh_ref,
              bh_ref, out_ref):
    z = agg_ref[0] + agg_ref[1] + (eps_ref[0, 0] - 1.0) * u2_ref[...] + b2a_ref[...]
    t = jax.nn.relu(z)
    h2 = jax.nn.relu(jnp.dot(t, W2b_ref[...], preferred_element_type=jnp.float32, precision=jax.lax.Precision.HIGHEST)
                     + b2b_ref[...])
    out_ref[...] = jnp.dot(h2, Wh_ref[...], preferred_element_type=jnp.float32, precision=jax.lax.Precision.HIGHEST) \
        + bh_ref[...]


def _tc2(eps2, u2p, agg2p, b2a4, W2b_bd, b2b4, Wh_bd, bh4):
    grid = (1,)
    return pl.pallas_call(
        _tc2_body,
        grid=grid,
        in_specs=[
            pl.BlockSpec(memory_space=pltpu.SMEM),
            pl.BlockSpec((N4, 128), lambda i: (0, 0)),
            pl.BlockSpec((NC, N4, 128), lambda i: (0, 0, 0)),
            pl.BlockSpec((1, 128), lambda i: (0, 0)),
            pl.BlockSpec((128, 128), lambda i: (0, 0)),
            pl.BlockSpec((1, 128), lambda i: (0, 0)),
            pl.BlockSpec((128, 16), lambda i: (0, 0)),
            pl.BlockSpec((1, 16), lambda i: (0, 0)),
        ],
        out_specs=pl.BlockSpec((N4, 16), lambda i: (0, 0)),
        out_shape=jax.ShapeDtypeStruct((N4, 16), jnp.float32),
    )(eps2, u2p, agg2p, b2a4, W2b_bd, b2b4, Wh_bd, bh4)


def kernel(x, edge_index, eps1, W1a, b1a, W1b, b1b, eps2, W2a, b2a, W2b, b2b,
           Wh, bh):
    ei = jnp.reshape(edge_index, (2 * E,))
    eps1_s = jnp.reshape(eps1, (1, 1))
    eps2_s = jnp.reshape(eps2, (1, 1))
    agg1 = _make_sc_agg(128)(x, ei)
    u2 = _tc1(eps1_s, x, agg1,
              W1a, jnp.reshape(b1a, (1, 128)),
              W1b, jnp.reshape(b1b, (1, 128)), W2a)
    u2p = jnp.reshape(u2, (N4, 128))
    agg2 = _make_sc_agg(32)(u2, ei)
    agg2p = jnp.reshape(agg2, (NC, N4, 128))
    W2b_bd = jnp.concatenate(
        [jnp.pad(W2b, ((32 * j, 96 - 32 * j), (0, 0))) for j in range(4)],
        axis=1)
    Wh_bd = jnp.concatenate(
        [jnp.pad(Wh, ((32 * j, 96 - 32 * j), (0, 0))) for j in range(4)],
        axis=1)
    outp = _tc2(eps2_s, u2p, agg2p,
                jnp.reshape(jnp.tile(b2a, 4), (1, 128)),
                W2b_bd,
                jnp.reshape(jnp.tile(b2b, 4), (1, 128)),
                Wh_bd,
                jnp.reshape(jnp.tile(bh, 4), (1, 16)))
    return jnp.reshape(outp, (N, 4))
